# baseline (speedup 1.0000x reference)
import functools

import jax
import jax.numpy as jnp
from jax import lax
from jax.experimental import pallas as pl
from jax.experimental.pallas import tpu as pltpu
from jax.experimental.pallas import tpu_sc as plsc

_NC = 2
_NS = 16
_L = 16
_NW = _NC * _NS
_CHUNK = 128
_BC = 65536


@functools.cache
def _row_extract(n_heads: int, card_x: int):
    grid = pl.cdiv(card_x, _BC)

    @functools.partial(
        pl.pallas_call,
        grid=(grid,),
        in_specs=[
            pl.BlockSpec((n_heads, _BC), lambda j: (0, j)),
            pl.BlockSpec(memory_space=pltpu.SMEM),
        ],
        out_specs=pl.BlockSpec((_BC,), lambda j: (j,)),
        out_shape=jax.ShapeDtypeStruct((card_x,), jnp.int32),
    )
    def row_extract(tab_ref, i_ref, out_ref):
        sel = lax.broadcasted_iota(jnp.int32, (n_heads, _BC), 0) == i_ref[0]
        out_ref[...] = jnp.sum(jnp.where(sel, tab_ref[...], 0), axis=0)

    return row_extract


@functools.cache
def _gather(card_x: int, batch: int):
    b_per_w = batch // _NW
    n_chunks = b_per_w // _CHUNK
    mesh = plsc.VectorSubcoreMesh(
        core_axis_name="c", subcore_axis_name="s",
        num_cores=_NC, num_subcores=_NS,
    )

    @functools.partial(
        pl.kernel,
        mesh=mesh,
        out_type=jax.ShapeDtypeStruct((batch,), jnp.int32),
        scratch_types=[
            pltpu.VMEM((n_chunks, _CHUNK), jnp.int32),
            pltpu.VMEM((b_per_w,), jnp.int32),
            pltpu.SemaphoreType.DMA,
        ],
    )
    def gather_kernel(row_hbm, idx_hbm, out_hbm, idx_v, res_v, sem):
        wid = lax.axis_index("s") * _NC + lax.axis_index("c")
        base = wid * b_per_w
        for c in range(n_chunks):
            pltpu.sync_copy(idx_hbm.at[pl.ds(base + c * _CHUNK, _CHUNK)],
                            idx_v.at[c])
        copies = [
            pltpu.async_copy(
                row_hbm.at[idx_v.at[c]],
                res_v.at[pl.ds(c * _CHUNK, _CHUNK)],
                sem,
            )
            for c in range(n_chunks)
        ]
        for cp in copies:
            cp.wait()
        pltpu.sync_copy(res_v, out_hbm.at[pl.ds(base, b_per_w)])

    return gather_kernel


def kernel(nuisances, i, idcs):
    n_heads, card_x = nuisances.shape
    i_arr = jnp.reshape(jnp.asarray(i, dtype=jnp.int32), (1,))
    row = _row_extract(n_heads, card_x)(nuisances, i_arr)
    return _gather(card_x, idcs.shape[0])(row, idcs.astype(jnp.int32))


# speedup vs baseline: 1.0359x; 1.0359x over previous
import functools

import jax
import jax.numpy as jnp
from jax import lax
from jax.experimental import pallas as pl
from jax.experimental.pallas import tpu as pltpu
from jax.experimental.pallas import tpu_sc as plsc

_NC = 2
_NS = 16
_L = 16
_NW = _NC * _NS
_CHUNK = 128
_BC = 65536


@functools.cache
def _row_extract(n_heads: int, card_x: int):
    grid = pl.cdiv(card_x, _BC)

    @functools.partial(
        pl.pallas_call,
        grid=(grid,),
        in_specs=[
            pl.BlockSpec((n_heads, _BC), lambda j: (0, j)),
            pl.BlockSpec(memory_space=pltpu.SMEM),
        ],
        out_specs=pl.BlockSpec((_BC,), lambda j: (j,)),
        out_shape=jax.ShapeDtypeStruct((card_x,), jnp.int32),
    )
    def row_extract(tab_ref, i_ref, out_ref):
        sel = lax.broadcasted_iota(jnp.int32, (n_heads, _BC), 0) == i_ref[0]
        out_ref[...] = jnp.sum(jnp.where(sel, tab_ref[...], 0), axis=0)

    return row_extract


@functools.cache
def _gather(card_x: int, batch: int):
    b_per_w = batch // _NW
    n_chunks = b_per_w // _CHUNK
    mesh = plsc.VectorSubcoreMesh(
        core_axis_name="c", subcore_axis_name="s",
        num_cores=_NC, num_subcores=_NS,
    )

    @functools.partial(
        pl.kernel,
        mesh=mesh,
        out_type=jax.ShapeDtypeStruct((batch,), jnp.int32),
        scratch_types=[
            pltpu.VMEM((b_per_w,), jnp.int32),
            pltpu.VMEM((b_per_w,), jnp.int32),
            pltpu.SemaphoreType.DMA,
        ],
    )
    def gather_kernel(row_hbm, idx_hbm, out_hbm, idx_v, res_v, sem):
        wid = lax.axis_index("s") * _NC + lax.axis_index("c")
        base = wid * b_per_w
        pltpu.sync_copy(idx_hbm.at[pl.ds(base, b_per_w)], idx_v)
        pltpu.async_copy(row_hbm.at[idx_v], res_v, sem).wait()
        pltpu.sync_copy(res_v, out_hbm.at[pl.ds(base, b_per_w)])

    return gather_kernel


def kernel(nuisances, i, idcs):
    n_heads, card_x = nuisances.shape
    i_arr = jnp.reshape(jnp.asarray(i, dtype=jnp.int32), (1,))
    row = _row_extract(n_heads, card_x)(nuisances, i_arr)
    return _gather(card_x, idcs.shape[0])(row, idcs.astype(jnp.int32))
